# baseline (speedup 1.0000x reference)
import functools

import jax
import jax.numpy as jnp
from jax import lax
from jax.experimental import pallas as pl
from jax.experimental.pallas import tpu as pltpu
from jax.experimental.pallas import tpu_sc as plsc

NC = 2
NS = 16
NW = NC * NS
GB = 64


def _make_sc_gather(batch, fields, dim):
    bpw = batch // NW
    n_groups = bpw // GB
    mesh = plsc.VectorSubcoreMesh(core_axis_name="c", subcore_axis_name="s")

    @functools.partial(
        pl.kernel,
        out_type=jax.ShapeDtypeStruct((batch, fields, dim), jnp.float32),
        mesh=mesh,
        scratch_types=[
            pltpu.VMEM((bpw, fields), jnp.int32),
            pltpu.VMEM((GB, fields, dim), jnp.float32),
            pltpu.VMEM((GB, fields, dim), jnp.float32),
            pltpu.SemaphoreType.DMA,
            pltpu.SemaphoreType.DMA,
            pltpu.SemaphoreType.DMA,
            pltpu.SemaphoreType.DMA,
        ],
        compiler_params=pltpu.CompilerParams(use_tc_tiling_on_sc=False),
    )
    def body(idx_hbm, tab_hbm, out_hbm, idx_v, rows0, rows1, g0, g1, o0, o1):
        wid = lax.axis_index("s") * NC + lax.axis_index("c")
        base = wid * bpw
        pltpu.sync_copy(idx_hbm.at[wid], idx_v)

        rows = (rows0, rows1)
        gsem = (g0, g1)
        osem = (o0, o1)

        def fire_gather(g, b):
            def fire4(i, carry):
                for u in range(4):
                    jj = i * 4 + u
                    pltpu.async_copy(
                        tab_hbm.at[idx_v.at[g * GB + jj]],
                        rows[b].at[jj],
                        gsem[b],
                    )
                return carry

            lax.fori_loop(0, GB // 4, fire4, 0)

        def drain_gather(b):
            pltpu.make_async_copy(out_hbm.at[pl.ds(0, GB)], rows[b],
                                  gsem[b]).wait()

        def fire_out(g, b):
            pltpu.async_copy(
                rows[b], out_hbm.at[pl.ds(base + g * GB, GB)], osem[b]
            )

        def wait_out(b):
            pltpu.make_async_copy(rows[b], out_hbm.at[pl.ds(0, GB)],
                                  osem[b]).wait()

        fire_gather(0, 0)

        def step(g, carry):
            b = g % 2

            def one(bb):
                drain_gather(bb)
                fire_out(g, bb)

                @pl.when(g + 1 < n_groups)
                def _():
                    @pl.when(g >= 1)
                    def _():
                        wait_out(1 - bb)
                    fire_gather(g + 1, 1 - bb)

            @pl.when(b == 0)
            def _():
                one(0)

            @pl.when(b == 1)
            def _():
                one(1)

            return carry

        lax.fori_loop(0, n_groups, step, 0)
        wait_out(0)
        wait_out(1)

    return body


def kernel(x, E):
    b, f = x.shape
    v, d = E.shape
    xf = x.astype(jnp.int32).reshape(NW, b // NW, f)
    return _make_sc_gather(b, f, d)(xf, E)


# speedup vs baseline: 1.0045x; 1.0045x over previous
import functools

import jax
import jax.numpy as jnp
from jax import lax
from jax.experimental import pallas as pl
from jax.experimental.pallas import tpu as pltpu
from jax.experimental.pallas import tpu_sc as plsc

NC = 2
NS = 16
NW = NC * NS
GB = 64


def _make_sc_gather(batch, fields, dim):
    bpw = batch // NW
    n_groups = bpw // GB
    mesh = plsc.VectorSubcoreMesh(core_axis_name="c", subcore_axis_name="s")

    @functools.partial(
        pl.kernel,
        out_type=jax.ShapeDtypeStruct((batch, fields, dim), jnp.float32),
        mesh=mesh,
        scratch_types=[
            pltpu.VMEM((bpw, fields), jnp.int32),
            pltpu.VMEM((GB, fields, dim), jnp.float32),
            pltpu.VMEM((GB, fields, dim), jnp.float32),
            pltpu.SemaphoreType.DMA,
            pltpu.SemaphoreType.DMA,
            pltpu.SemaphoreType.DMA,
            pltpu.SemaphoreType.DMA,
        ],
        compiler_params=pltpu.CompilerParams(use_tc_tiling_on_sc=False),
    )
    def body(idx_hbm, tab_hbm, out_hbm, idx_v, rows0, rows1, g0, g1, o0, o1):
        wid = lax.axis_index("s") * NC + lax.axis_index("c")
        base = wid * bpw
        pltpu.sync_copy(idx_hbm.at[wid], idx_v)

        rows = (rows0, rows1)
        gsem = (g0, g1)
        osem = (o0, o1)

        def fire_gather(g, b):
            def fire4(i, carry):
                for u in range(4):
                    jj = i * 4 + u
                    pltpu.async_copy(
                        tab_hbm.at[idx_v.at[g * GB + jj]],
                        rows[b].at[jj],
                        gsem[b],
                    )
                return carry

            lax.fori_loop(0, GB // 4, fire4, 0)

        def drain_gather(b):
            pltpu.make_async_copy(out_hbm.at[pl.ds(0, GB)], rows[b],
                                  gsem[b]).wait()

        def fire_out(g, b):
            pltpu.async_copy(
                rows[b], out_hbm.at[pl.ds(base + g * GB, GB)], osem[b]
            )

        def wait_out(b):
            pltpu.make_async_copy(rows[b], out_hbm.at[pl.ds(0, GB)],
                                  osem[b]).wait()

        fire_gather(0, 0)

        def step(g, carry):
            b = g % 2

            def one(bb):
                drain_gather(bb)
                fire_out(g, bb)

                @pl.when(g + 1 < n_groups)
                def _():
                    @pl.when(g >= 1)
                    def _():
                        wait_out(1 - bb)
                    fire_gather(g + 1, 1 - bb)

            @pl.when(b == 0)
            def _():
                one(0)

            @pl.when(b == 1)
            def _():
                one(1)

            return carry

        lax.fori_loop(0, n_groups, step, 0)
        wait_out(0)
        wait_out(1)

    return body


def kernel(x, E):
    b, f = x.shape
    v, d = E.shape
    h = b // 2
    mk = _make_sc_gather(h, f, d)
    outs = []
    for i in range(2):
        xh = x[i * h:(i + 1) * h].astype(jnp.int32).reshape(NW, h // NW, f)
        outs.append(mk(xh, E))
    return jnp.concatenate(outs, axis=0)
